# baseline (speedup 1.0000x reference)
import jax
import jax.numpy as jnp
from jax.experimental import pallas as pl
from jax.experimental.pallas import tpu as pltpu

_TT = 512
_CK = 2048


def _rvq_body(x_ref, w_ref, q_ref, c_ref, loss_ref):
    nq, k, d = w_ref.shape
    tt = x_ref.shape[0]
    t = pl.program_id(0)
    x = x_ref[...]
    col_iota = jax.lax.broadcasted_iota(jnp.int32, (tt, nq), 1)
    iota = jax.lax.broadcasted_iota(jnp.int32, (tt, _CK), 1)

    def chunk_step(c, carry):
        run_min, run_arg, run_row, r, r2, q = carry
        wc = w_ref[q, pl.ds(c * _CK, _CK), :]
        w2 = jnp.sum(wc * wc, axis=1)[None, :]
        e = jax.lax.dot_general(
            r, wc, (((1,), (1,)), ((), ())),
            preferred_element_type=jnp.float32)
        d2 = (r2 - 2.0 * e) + w2
        lmin = jnp.min(d2, axis=1, keepdims=True)
        larg = jnp.min(jnp.where(d2 == lmin, iota, k),
                       axis=1, keepdims=True)
        onehot = (iota == larg).astype(jnp.float32)
        lrow = jax.lax.dot_general(
            onehot, wc, (((1,), (0,)), ((), ())),
            precision=jax.lax.Precision.HIGHEST,
            preferred_element_type=jnp.float32)
        better = lmin < run_min
        run_arg = jnp.where(better, larg + c * _CK, run_arg)
        run_row = jnp.where(better, lrow, run_row)
        run_min = jnp.minimum(run_min, lmin)
        return run_min, run_arg, run_row, r, r2, q

    def quant_step(q, carry):
        r, codes_mat = carry
        r2 = jnp.sum(r * r, axis=1, keepdims=True)
        run_min = jnp.full((tt, 1), jnp.inf, jnp.float32)
        run_arg = jnp.zeros((tt, 1), jnp.int32)
        run_row = jnp.zeros((tt, d), jnp.float32)
        run_min, run_arg, run_row, _, _, _ = jax.lax.fori_loop(
            0, k // _CK, chunk_step,
            (run_min, run_arg, run_row, r, r2, q))
        codes_mat = codes_mat + jnp.where(col_iota == q, run_arg, 0)
        return r - run_row, codes_mat

    r, codes_mat = jax.lax.fori_loop(
        0, nq, quant_step, (x, jnp.zeros((tt, nq), jnp.int32)))
    q_ref[...] = x - r
    c_ref[...] = codes_mat
    psum = jnp.sum(r * r).reshape(1, 1)

    @pl.when(t == 0)
    def _():
        loss_ref[...] = jnp.zeros((1, 1), jnp.float32)

    loss_ref[...] += psum


def kernel(input, weight, running_mean, code_count):
    b, t, d = input.shape
    nq, k, _ = weight.shape
    n = b * t
    x = input.reshape(n, d).astype(jnp.float32)
    quant, codes, loss = pl.pallas_call(
        _rvq_body,
        grid=(n // _TT,),
        in_specs=[
            pl.BlockSpec((_TT, d), lambda i: (i, 0)),
            pl.BlockSpec((nq, k, d), lambda i: (0, 0, 0)),
        ],
        out_specs=[
            pl.BlockSpec((_TT, d), lambda i: (i, 0)),
            pl.BlockSpec((_TT, nq), lambda i: (i, 0)),
            pl.BlockSpec((1, 1), lambda i: (0, 0)),
        ],
        out_shape=[
            jax.ShapeDtypeStruct((n, d), jnp.float32),
            jax.ShapeDtypeStruct((n, nq), jnp.int32),
            jax.ShapeDtypeStruct((1, 1), jnp.float32),
        ],
        compiler_params=pltpu.CompilerParams(
            dimension_semantics=("arbitrary",),
        ),
    )(x, weight)
    quantized = quant.reshape(b, t, d)
    codes_out = codes.reshape(b, t, nq)
    commitment_loss = (loss[0, 0] / jnp.float32(n * d))
    return quantized, codes_out, commitment_loss


# speedup vs baseline: 2.1087x; 2.1087x over previous
import jax
import jax.numpy as jnp
from jax.experimental import pallas as pl
from jax.experimental.pallas import tpu as pltpu

_TT = 512
_CK = 2048


def _rvq_body(x_ref, wt_ref, w2_ref, wsplit_ref, q_ref, c_ref, loss_ref):
    nq, d, k = wt_ref.shape
    tt = x_ref.shape[0]
    t = pl.program_id(0)
    x = x_ref[...]
    col_iota = jax.lax.broadcasted_iota(jnp.int32, (tt, nq), 1)
    iota_f = jax.lax.broadcasted_iota(jnp.int32, (tt, _CK), 1).astype(jnp.float32)

    r = x
    codes_mat = jnp.zeros((tt, nq), jnp.int32)
    for q in range(nq):
        r2 = jnp.sum(r * r, axis=1, keepdims=True)

        def chunk_step(c, carry, r=r, r2=r2, q=q):
            run_min, run_arg, run_row = carry
            wct = wt_ref[q, :, pl.ds(c * _CK, _CK)]
            w2 = w2_ref[q, 0:1, pl.ds(c * _CK, _CK)]
            e = jax.lax.dot_general(
                r, wct, (((1,), (0,)), ((), ())),
                preferred_element_type=jnp.float32)
            d2 = (r2 - 2.0 * e) + w2
            lmin = jnp.min(d2, axis=1, keepdims=True)
            sel = jnp.where(d2 == lmin, iota_f, jnp.float32(_CK))
            larg_f = jnp.min(sel, axis=1, keepdims=True)
            onehot = (iota_f == larg_f).astype(jnp.bfloat16)
            parts = jax.lax.dot_general(
                onehot, wsplit_ref[q, :, pl.ds(c * _CK, _CK)],
                (((1,), (1,)), ((), ())),
                preferred_element_type=jnp.float32)
            lrow = (parts[:, 0:d] + parts[:, d:2 * d]) + parts[:, 2 * d:3 * d]
            better = lmin < run_min
            larg = larg_f.astype(jnp.int32) + c * _CK
            run_arg = jnp.where(better, larg, run_arg)
            run_row = jnp.where(better, lrow, run_row)
            run_min = jnp.minimum(run_min, lmin)
            return run_min, run_arg, run_row

        run_min, run_arg, run_row = jax.lax.fori_loop(
            0, k // _CK, chunk_step,
            (jnp.full((tt, 1), jnp.inf, jnp.float32),
             jnp.zeros((tt, 1), jnp.int32),
             jnp.zeros((tt, d), jnp.float32)))
        codes_mat = codes_mat + jnp.where(col_iota == q, run_arg, 0)
        r = r - run_row
    q_ref[...] = x - r
    c_ref[...] = codes_mat
    psum = jnp.sum(r * r).reshape(1, 1)

    @pl.when(t == 0)
    def _():
        loss_ref[...] = jnp.zeros((1, 1), jnp.float32)

    loss_ref[...] += psum


def kernel(input, weight, running_mean, code_count):
    b, t, d = input.shape
    nq, k, _ = weight.shape
    n = b * t
    x = input.reshape(n, d).astype(jnp.float32)
    wt = weight.transpose(0, 2, 1)
    w2 = jnp.sum(weight * weight, axis=-1).reshape(nq, 1, k)
    def _trunc_bf16(v):
        bits = jax.lax.bitcast_convert_type(v, jnp.uint32)
        return jax.lax.bitcast_convert_type(
            bits & jnp.uint32(0xFFFF0000), jnp.float32)
    hi = _trunc_bf16(weight)
    rem1 = weight - hi
    mid = _trunc_bf16(rem1)
    lo = rem1 - mid
    w_hi = hi.astype(jnp.bfloat16)
    w_mid = mid.astype(jnp.bfloat16)
    w_lo = lo.astype(jnp.bfloat16)
    wsplit = jnp.concatenate(
        [w_hi.transpose(0, 2, 1), w_mid.transpose(0, 2, 1),
         w_lo.transpose(0, 2, 1)], axis=1)
    quant, codes, loss = pl.pallas_call(
        _rvq_body,
        grid=(n // _TT,),
        in_specs=[
            pl.BlockSpec((_TT, d), lambda i: (i, 0)),
            pl.BlockSpec((nq, d, k), lambda i: (0, 0, 0)),
            pl.BlockSpec((nq, 1, k), lambda i: (0, 0, 0)),
            pl.BlockSpec((nq, 3 * d, k), lambda i: (0, 0, 0)),
        ],
        out_specs=[
            pl.BlockSpec((_TT, d), lambda i: (i, 0)),
            pl.BlockSpec((_TT, nq), lambda i: (i, 0)),
            pl.BlockSpec((1, 1), lambda i: (0, 0)),
        ],
        out_shape=[
            jax.ShapeDtypeStruct((n, d), jnp.float32),
            jax.ShapeDtypeStruct((n, nq), jnp.int32),
            jax.ShapeDtypeStruct((1, 1), jnp.float32),
        ],
        compiler_params=pltpu.CompilerParams(
            dimension_semantics=("arbitrary",),
        ),
    )(x, wt, w2, wsplit)
    quantized = quant.reshape(b, t, d)
    codes_out = codes.reshape(b, t, nq)
    commitment_loss = (loss[0, 0] / jnp.float32(n * d))
    return quantized, codes_out, commitment_loss
